# Initial kernel scaffold; baseline (speedup 1.0000x reference)
import jax
import jax.numpy as jnp
from jax.experimental import pallas as pl


def kernel(x_nchw, w1, w2, g1, b1, g2, b2):
    raise NotImplementedError("write your pallas kernel here")


# speedup vs baseline: 1.9014x; 1.9014x over previous
import jax
import jax.numpy as jnp
from jax.experimental import pallas as pl
from jax.experimental.pallas import tpu as pltpu

EPS = 1e-5
_VMEM_LIMIT = 64 * 1024 * 1024


def _rowpad(a, c, w):
    zrow = jnp.zeros((c, 1, w), a.dtype)
    return jnp.concatenate([zrow, a, zrow], axis=1)


def _conv3x3(xrp, w_ref, c, h, w):
    zcol = jnp.zeros((c, h, 1), xrp.dtype)
    acc = None
    for kh in range(3):
        xs = xrp[:, kh:kh + h, :]
        for kw in range(3):
            if kw == 0:
                tap = jnp.concatenate([zcol, xs[:, :, :w - 1]], axis=2)
            elif kw == 2:
                tap = jnp.concatenate([xs[:, :, 1:], zcol], axis=2)
            else:
                tap = xs
            d = jnp.dot(w_ref[kh * 3 + kw], tap.reshape(c, h * w),
                        preferred_element_type=jnp.float32)
            acc = d if acc is None else acc + d
    return acc


def _stats2(y):
    s = jnp.sum(y, axis=1, keepdims=True)
    ss = jnp.sum(y * y, axis=1, keepdims=True)
    return jnp.concatenate([s, ss], axis=1)[None]


def _conv1_kernel(x_ref, w_ref, y_ref, st_ref):
    _, c, h, w = x_ref.shape
    xb = x_ref[0].astype(jnp.bfloat16)
    y = _conv3x3(_rowpad(xb, c, w), w_ref, c, h, w)
    y_ref[...] = y.reshape(1, c, h, w).astype(jnp.bfloat16)
    st_ref[...] = _stats2(y)


def _bn_relu_conv2_kernel(y1_ref, w_ref, sc_ref, sh_ref, y_ref, st_ref):
    _, c, h, w = y1_ref.shape
    sc = sc_ref[...].reshape(c, 1, 1)
    sh = sh_ref[...].reshape(c, 1, 1)
    r = jnp.maximum(y1_ref[0] * sc + sh, 0.0).astype(jnp.bfloat16)
    y = _conv3x3(_rowpad(r, c, w), w_ref, c, h, w)
    y_ref[...] = y.reshape(1, c, h, w).astype(jnp.bfloat16)
    st_ref[...] = _stats2(y)


def _bn_add_relu_kernel(y2_ref, x_ref, sc_ref, sh_ref, o_ref):
    c = o_ref.shape[1]
    sc = sc_ref[...].reshape(1, c, 1, 1)
    sh = sh_ref[...].reshape(1, c, 1, 1)
    o_ref[...] = jnp.maximum(y2_ref[...] * sc + sh + x_ref[...], 0.0)


def _finalize_bn(st, gamma, beta, count):
    s = jnp.sum(st[:, :, 0], axis=0)
    ss = jnp.sum(st[:, :, 1], axis=0)
    mean = s / count
    var = jnp.maximum(ss / count - mean * mean, 0.0)
    scale = gamma.reshape(-1) * jax.lax.rsqrt(var + EPS)
    shift = beta.reshape(-1) - mean * scale
    return scale.reshape(-1, 1), shift.reshape(-1, 1)


@jax.jit
def kernel(x_nchw, w1, w2, g1, b1, g2, b2):
    n, c, h, w = x_nchw.shape
    count = float(n * h * w)
    w1t = jnp.transpose(w1, (0, 2, 1)).astype(jnp.bfloat16)
    w2t = jnp.transpose(w2, (0, 2, 1)).astype(jnp.bfloat16)

    img_spec = pl.BlockSpec((1, c, h, w), lambda i: (i, 0, 0, 0))
    w_spec = pl.BlockSpec((9, c, c), lambda i: (0, 0, 0))
    vec_spec = pl.BlockSpec((c, 1), lambda i: (0, 0))
    st_spec = pl.BlockSpec((1, c, 2), lambda i: (i, 0, 0))
    act_bf16 = jax.ShapeDtypeStruct((n, c, h, w), jnp.bfloat16)
    st_shape = jax.ShapeDtypeStruct((n, c, 2), jnp.float32)
    cparams = pltpu.CompilerParams(dimension_semantics=("parallel",),
                                   vmem_limit_bytes=_VMEM_LIMIT)

    y1, st1 = pl.pallas_call(
        _conv1_kernel, grid=(n,),
        in_specs=[img_spec, w_spec],
        out_specs=(img_spec, st_spec),
        out_shape=(act_bf16, st_shape),
        compiler_params=cparams)(x_nchw, w1t)
    sc1, sh1 = _finalize_bn(st1, g1, b1, count)

    y2, st2 = pl.pallas_call(
        _bn_relu_conv2_kernel, grid=(n,),
        in_specs=[img_spec, w_spec, vec_spec, vec_spec],
        out_specs=(img_spec, st_spec),
        out_shape=(act_bf16, st_shape),
        compiler_params=cparams)(y1, w2t, sc1, sh1)
    sc2, sh2 = _finalize_bn(st2, g2, b2, count)

    gh3 = 4
    blk3 = pl.BlockSpec((1, c, h // gh3, w), lambda i, j: (i, 0, j, 0))
    vec3 = pl.BlockSpec((c, 1), lambda i, j: (0, 0))
    out = pl.pallas_call(
        _bn_add_relu_kernel, grid=(n, gh3),
        in_specs=[blk3, blk3, vec3, vec3],
        out_specs=blk3,
        out_shape=jax.ShapeDtypeStruct((n, c, h, w), jnp.float32),
        compiler_params=pltpu.CompilerParams(
            dimension_semantics=("parallel", "arbitrary"),
            vmem_limit_bytes=_VMEM_LIMIT))(y2, x_nchw, sc2, sh2)
    return out


# speedup vs baseline: 2.6619x; 1.4000x over previous
import functools

import jax
import jax.numpy as jnp
from jax.experimental import pallas as pl
from jax.experimental.pallas import tpu as pltpu

EPS = 1e-5
_VMEM_LIMIT = 64 * 1024 * 1024


def _conv3x3_flat(xf, wc_ref, mk_ref, c, hw, w):
    z = jnp.zeros((c, 2 * w), xf.dtype)
    zc = jnp.zeros((c, 1), xf.dtype)
    pf = jnp.concatenate([z, xf, z], axis=1)
    base_m1 = jnp.concatenate([zc, pf[:, :-1]], axis=1) * mk_ref[0:1, :]
    base_p1 = jnp.concatenate([pf[:, 1:], zc], axis=1) * mk_ref[1:2, :]
    bases = {-1: base_m1, 0: pf, 1: base_p1}

    def tap(k):
        dh, dw = k // 3 - 1, k % 3 - 1
        return bases[dw][:, 2 * w + dh * w: 2 * w + dh * w + hw]

    acc = None
    for j in range(4):
        pair = jnp.concatenate([tap(2 * j), tap(2 * j + 1)], axis=0)
        d = jnp.dot(wc_ref[:, 2 * c * j: 2 * c * (j + 1)], pair,
                    preferred_element_type=jnp.float32)
        acc = d if acc is None else acc + d
    d = jnp.dot(wc_ref[:, 8 * c: 9 * c], tap(8),
                preferred_element_type=jnp.float32)
    return acc + d


def _stats2(y):
    s = jnp.sum(y, axis=1, keepdims=True)
    ss = jnp.sum(y * y, axis=1, keepdims=True)
    return jnp.concatenate([s, ss], axis=1)[None]


def _conv1_kernel(w_s, x_ref, wc_ref, mk_ref, y_ref, st_ref):
    _, c, hw = x_ref.shape
    xb = x_ref[0].astype(jnp.bfloat16)
    y = _conv3x3_flat(xb, wc_ref, mk_ref, c, hw, w_s)
    y_ref[...] = y[None].astype(jnp.bfloat16)
    st_ref[...] = _stats2(y)


def _bn_relu_conv2_kernel(w_s, y1_ref, wc_ref, mk_ref, sc_ref, sh_ref,
                          y_ref, st_ref):
    _, c, hw = y1_ref.shape
    sc = sc_ref[...].astype(jnp.bfloat16)
    sh = sh_ref[...].astype(jnp.bfloat16)
    r = jnp.maximum(y1_ref[0] * sc + sh, 0.0)
    y = _conv3x3_flat(r, wc_ref, mk_ref, c, hw, w_s)
    y_ref[...] = y[None].astype(jnp.bfloat16)
    st_ref[...] = _stats2(y)


def _bn_add_relu_kernel(y2_ref, x_ref, sc_ref, sh_ref, o_ref):
    sc = sc_ref[...][None]
    sh = sh_ref[...][None]
    o_ref[...] = jnp.maximum(y2_ref[...] * sc + sh + x_ref[...], 0.0)


def _finalize_bn(st, gamma, beta, count):
    s = jnp.sum(st[:, :, 0], axis=0)
    ss = jnp.sum(st[:, :, 1], axis=0)
    mean = s / count
    var = jnp.maximum(ss / count - mean * mean, 0.0)
    scale = gamma.reshape(-1) * jax.lax.rsqrt(var + EPS)
    shift = beta.reshape(-1) - mean * scale
    return scale.reshape(-1, 1), shift.reshape(-1, 1)


@jax.jit
def kernel(x_nchw, w1, w2, g1, b1, g2, b2):
    n, c, h, w = x_nchw.shape
    hw = h * w
    count = float(n * hw)
    xf = x_nchw.reshape(n, c, hw)
    w1c = jnp.transpose(w1, (2, 0, 1)).reshape(c, 9 * c).astype(jnp.bfloat16)
    w2c = jnp.transpose(w2, (2, 0, 1)).reshape(c, 9 * c).astype(jnp.bfloat16)
    lane = jnp.arange(hw + 4 * w, dtype=jnp.int32) % w
    masks = jnp.stack([(lane != 0), (lane != w - 1)]).astype(jnp.bfloat16)

    img_spec = pl.BlockSpec((1, c, hw), lambda i: (i, 0, 0))
    w_spec = pl.BlockSpec((c, 9 * c), lambda i: (0, 0))
    mk_spec = pl.BlockSpec((2, hw + 4 * w), lambda i: (0, 0))
    vec_spec = pl.BlockSpec((c, 1), lambda i: (0, 0))
    st_spec = pl.BlockSpec((1, c, 2), lambda i: (i, 0, 0))
    act_bf16 = jax.ShapeDtypeStruct((n, c, hw), jnp.bfloat16)
    st_shape = jax.ShapeDtypeStruct((n, c, 2), jnp.float32)
    cparams = pltpu.CompilerParams(dimension_semantics=("parallel",),
                                   vmem_limit_bytes=_VMEM_LIMIT)

    y1, st1 = pl.pallas_call(
        functools.partial(_conv1_kernel, w),
        grid=(n,),
        in_specs=[img_spec, w_spec, mk_spec],
        out_specs=(img_spec, st_spec),
        out_shape=(act_bf16, st_shape),
        compiler_params=cparams)(xf, w1c, masks)
    sc1, sh1 = _finalize_bn(st1, g1, b1, count)

    y2, st2 = pl.pallas_call(
        functools.partial(_bn_relu_conv2_kernel, w),
        grid=(n,),
        in_specs=[img_spec, w_spec, mk_spec, vec_spec, vec_spec],
        out_specs=(img_spec, st_spec),
        out_shape=(act_bf16, st_shape),
        compiler_params=cparams)(y1, w2c, masks, sc1, sh1)
    sc2, sh2 = _finalize_bn(st2, g2, b2, count)

    gh3 = 4
    blk3 = pl.BlockSpec((1, c, hw // gh3), lambda i, j: (i, 0, j))
    vec3 = pl.BlockSpec((c, 1), lambda i, j: (0, 0))
    out = pl.pallas_call(
        _bn_add_relu_kernel, grid=(n, gh3),
        in_specs=[blk3, blk3, vec3, vec3],
        out_specs=blk3,
        out_shape=jax.ShapeDtypeStruct((n, c, hw), jnp.float32),
        compiler_params=pltpu.CompilerParams(
            dimension_semantics=("parallel", "arbitrary"),
            vmem_limit_bytes=_VMEM_LIMIT))(y2, xf, sc2, sh2)
    return out.reshape(n, c, h, w)


# speedup vs baseline: 4.3253x; 1.6249x over previous
import functools

import jax
import jax.numpy as jnp
from jax.experimental import pallas as pl
from jax.experimental.pallas import tpu as pltpu

EPS = 1e-5
_VMEM_LIMIT = 64 * 1024 * 1024


def _conv3x3_flat(xf, wc_ref, mk_ref, c, hw, w):
    z = jnp.zeros((c, 2 * w), xf.dtype)
    zc = jnp.zeros((c, 1), xf.dtype)
    pf = jnp.concatenate([z, xf, z], axis=1)
    base_m1 = jnp.concatenate([zc, pf[:, :-1]], axis=1) * mk_ref[0:1, :]
    base_p1 = jnp.concatenate([pf[:, 1:], zc], axis=1) * mk_ref[1:2, :]
    b = jnp.concatenate([base_m1, pf, base_p1], axis=0)

    acc = None
    for g in range(3):
        d = jnp.dot(wc_ref[:, 3 * c * g: 3 * c * (g + 1)],
                    b[:, (g + 1) * w: (g + 1) * w + hw],
                    preferred_element_type=jnp.float32)
        acc = d if acc is None else acc + d
    return acc


def _stats2(y):
    s = jnp.sum(y, axis=1, keepdims=True)
    ss = jnp.sum(y * y, axis=1, keepdims=True)
    return jnp.concatenate([s, ss], axis=1)[None]


def _conv1_kernel(x_ref, wc_ref, mk_ref, y_ref, st_ref):
    _, c, h, w = x_ref.shape
    hw = h * w
    xb = x_ref[0].astype(jnp.bfloat16).reshape(c, hw)
    y = _conv3x3_flat(xb, wc_ref, mk_ref, c, hw, w)
    y_ref[...] = y[None].astype(jnp.bfloat16)
    st_ref[...] = _stats2(y)


def _bn_relu_conv2_kernel(w_s, y1_ref, wc_ref, mk_ref, sc_ref, sh_ref,
                          y_ref, st_ref):
    _, c, hw = y1_ref.shape
    sc = sc_ref[...].astype(jnp.bfloat16)
    sh = sh_ref[...].astype(jnp.bfloat16)
    r = jnp.maximum(y1_ref[0] * sc + sh, 0.0)
    y = _conv3x3_flat(r, wc_ref, mk_ref, c, hw, w_s)
    y_ref[...] = y[None].astype(jnp.bfloat16)
    st_ref[...] = _stats2(y)


def _bn_add_relu_kernel(y2_ref, x_ref, sc_ref, sh_ref, o_ref):
    _, c, th, w = o_ref.shape
    sc = sc_ref[...].reshape(1, c, 1, 1)
    sh = sh_ref[...].reshape(1, c, 1, 1)
    y2 = y2_ref[0].reshape(c, th, w)[None]
    o_ref[...] = jnp.maximum(y2 * sc + sh + x_ref[...], 0.0)


def _finalize_bn(st, gamma, beta, count):
    s = jnp.sum(st[:, :, 0], axis=0)
    ss = jnp.sum(st[:, :, 1], axis=0)
    mean = s / count
    var = jnp.maximum(ss / count - mean * mean, 0.0)
    scale = gamma.reshape(-1) * jax.lax.rsqrt(var + EPS)
    shift = beta.reshape(-1) - mean * scale
    return scale.reshape(-1, 1), shift.reshape(-1, 1)


@jax.jit
def kernel(x_nchw, w1, w2, g1, b1, g2, b2):
    n, c, h, w = x_nchw.shape
    hw = h * w
    count = float(n * hw)
    w1c = jnp.transpose(w1, (2, 0, 1)).reshape(c, 9 * c).astype(jnp.bfloat16)
    w2c = jnp.transpose(w2, (2, 0, 1)).reshape(c, 9 * c).astype(jnp.bfloat16)
    lane = jnp.arange(hw + 4 * w, dtype=jnp.int32) % w
    masks = jnp.stack([(lane != 0), (lane != w - 1)]).astype(jnp.bfloat16)

    img4_spec = pl.BlockSpec((1, c, h, w), lambda i: (i, 0, 0, 0))
    imgf_spec = pl.BlockSpec((1, c, hw), lambda i: (i, 0, 0))
    w_spec = pl.BlockSpec((c, 9 * c), lambda i: (0, 0))
    mk_spec = pl.BlockSpec((2, hw + 4 * w), lambda i: (0, 0))
    vec_spec = pl.BlockSpec((c, 1), lambda i: (0, 0))
    st_spec = pl.BlockSpec((1, c, 2), lambda i: (i, 0, 0))
    act_bf16 = jax.ShapeDtypeStruct((n, c, hw), jnp.bfloat16)
    st_shape = jax.ShapeDtypeStruct((n, c, 2), jnp.float32)
    cparams = pltpu.CompilerParams(dimension_semantics=("parallel",),
                                   vmem_limit_bytes=_VMEM_LIMIT)

    y1, st1 = pl.pallas_call(
        _conv1_kernel, grid=(n,),
        in_specs=[img4_spec, w_spec, mk_spec],
        out_specs=(imgf_spec, st_spec),
        out_shape=(act_bf16, st_shape),
        compiler_params=cparams)(x_nchw, w1c, masks)
    sc1, sh1 = _finalize_bn(st1, g1, b1, count)

    y2, st2 = pl.pallas_call(
        functools.partial(_bn_relu_conv2_kernel, w),
        grid=(n,),
        in_specs=[imgf_spec, w_spec, mk_spec, vec_spec, vec_spec],
        out_specs=(imgf_spec, st_spec),
        out_shape=(act_bf16, st_shape),
        compiler_params=cparams)(y1, w2c, masks, sc1, sh1)
    sc2, sh2 = _finalize_bn(st2, g2, b2, count)

    gh3 = 4
    blkf3 = pl.BlockSpec((1, c, hw // gh3), lambda i, j: (i, 0, j))
    blk43 = pl.BlockSpec((1, c, h // gh3, w), lambda i, j: (i, 0, j, 0))
    vec3 = pl.BlockSpec((c, 1), lambda i, j: (0, 0))
    out = pl.pallas_call(
        _bn_add_relu_kernel, grid=(n, gh3),
        in_specs=[blkf3, blk43, vec3, vec3],
        out_specs=blk43,
        out_shape=jax.ShapeDtypeStruct((n, c, h, w), jnp.float32),
        compiler_params=pltpu.CompilerParams(
            dimension_semantics=("parallel", "arbitrary"),
            vmem_limit_bytes=_VMEM_LIMIT))(y2, x_nchw, sc2, sh2)
    return out
